# Initial kernel scaffold; baseline (speedup 1.0000x reference)
import jax
import jax.numpy as jnp
from jax.experimental import pallas as pl


def kernel(X, edge_index, W_w, b_w, W_a, b_a, W_r, b_r):
    raise NotImplementedError("write your pallas kernel here")


# speedup vs baseline: 10.7685x; 10.7685x over previous
import functools

import jax
import jax.numpy as jnp
from jax import lax
from jax.experimental import pallas as pl
from jax.experimental.pallas import tpu as pltpu
from jax.experimental.pallas import tpu_sc as plsc

N = 10000
E = 160000
B = 2
D = 128
H = 64
ALPHA = 0.2
EPS = 9e-15

NP = 10240
ROWS_T = NP // 16

NSTG = 80
SR = 25
SW = 80

WW = 64
CROWS = 224

OCH = 64


def _tc_body(x_ref, ww_ref, bw_ref, wa_ref, ba_ref, wr_ref, br_ref,
             wh_ref, s_ref, res_ref):
    x = x_ref[0]
    wh = lax.dot_general(x, ww_ref[...], (((1,), (1,)), ((), ())),
                         preferred_element_type=jnp.float32) + bw_ref[...]
    wh = jnp.where(wh >= 0, wh, ALPHA * wh)
    res = lax.dot_general(x, wr_ref[...], (((1,), (1,)), ((), ())),
                          preferred_element_type=jnp.float32) + br_ref[...]
    sboth = lax.dot_general(wh, wa_ref[...], (((1,), (0,)), ((), ())),
                            preferred_element_type=jnp.float32)
    wh_ref[0, 0] = wh[:, :H]
    wh_ref[0, 1] = wh[:, H:]
    res_ref[0, 0] = res[:, :H]
    res_ref[0, 1] = res[:, H:]
    s_ref[0, 0] = sboth[:, 0:1] + ba_ref[0, 0]
    s_ref[0, 1] = sboth[:, 1:2]


_BN = 1024


def _tc_call(xp, ww, bw2, wa2, ba2, wr, br2):
    grid = (B, NP // _BN)
    return pl.pallas_call(
        _tc_body,
        grid=grid,
        in_specs=[
            pl.BlockSpec((1, _BN, D), lambda b, i: (b, i, 0)),
            pl.BlockSpec((D, D), lambda b, i: (0, 0)),
            pl.BlockSpec((1, D), lambda b, i: (0, 0)),
            pl.BlockSpec((D, D), lambda b, i: (0, 0)),
            pl.BlockSpec((1, 1), lambda b, i: (0, 0)),
            pl.BlockSpec((D, D), lambda b, i: (0, 0)),
            pl.BlockSpec((1, D), lambda b, i: (0, 0)),
        ],
        out_specs=[
            pl.BlockSpec((1, 2, _BN, H), lambda b, i: (b, 0, i, 0)),
            pl.BlockSpec((1, 2, _BN, 1), lambda b, i: (b, 0, i, 0)),
            pl.BlockSpec((1, 2, _BN, H), lambda b, i: (b, 0, i, 0)),
        ],
        out_shape=[
            jax.ShapeDtypeStruct((B, 2, NP, H), jnp.float32),
            jax.ShapeDtypeStruct((B, 2, NP, 1), jnp.float32),
            jax.ShapeDtypeStruct((B, 2, NP, H), jnp.float32),
        ],
    )(xp, ww, bw2, wa2, ba2, wr, br2)


_mesh = plsc.VectorSubcoreMesh(core_axis_name="c", subcore_axis_name="s")


def _sc_body(edge_ref, wh_ref, s_ref, res_ref, out_ref,
             sbuf, dbuf, clsrc, cdst, s1_loc, s2_t, e_buf, rows,
             h_own, rs_own, nbuf_r, obuf, inv_buf):
    _DBG_WIN = True
    c = lax.axis_index("c")
    t = lax.axis_index("s")
    lo = t * ROWS_T
    z16 = jnp.zeros((16,), jnp.float32)
    zi16 = jnp.zeros((16,), jnp.int32)
    iota = lax.iota(jnp.int32, 16)
    mask0 = iota == 0

    def _zh(i, _):
        plsc.store_scatter(h_own, [i * 16 + iota], z16)
        return 0
    lax.fori_loop(0, (ROWS_T * H) // 16, _zh, 0)
    for k in range((ROWS_T + 16) // 16):
        rs_own[k * 16:(k + 1) * 16] = z16

    def _stage(g, off):
        pltpu.sync_copy(edge_ref.at[0, g], sbuf)
        pltpu.sync_copy(edge_ref.at[1, g], dbuf)

        def srow(i, off):
            for j in range(SW // 16):
                sv = sbuf[i, j * 16:(j + 1) * 16]
                dv = dbuf[i, j * 16:(j + 1) * 16]
                m = (sv >= lo) & (sv < lo + ROWS_T)
                pm = plsc.cumsum(m.astype(jnp.int32))
                idx = off + pm - 1
                plsc.store_scatter(clsrc, [idx], sv - lo, mask=m)
                plsc.store_scatter(cdst, [idx >> 6, idx & 63], dv, mask=m)
                off = off + pm[15]
            return off
        return lax.fori_loop(0, SR, srow, off)

    m_cnt = lax.fori_loop(0, NSTG, _stage, jnp.int32(0))
    for k in range(6):
        fidx = m_cnt + k * 16 + iota
        plsc.store_scatter(clsrc, [fidx], zi16)
        plsc.store_scatter(cdst, [fidx >> 6, fidx & 63], zi16)
    nwin = (m_cnt + (WW - 1)) >> 6

    for b in range(B):
        pltpu.sync_copy(s_ref.at[b, 0, pl.ds(lo, ROWS_T)], s1_loc)
        pltpu.sync_copy(s_ref.at[b, 1], s2_t)
        wh_b = wh_ref.at[b, c]

        def win(w, _):
            for j in range(WW // 16):
                ji = j * 16 + iota
                dv16 = plsc.load_gather(
                    cdst, [jnp.full((16,), w, jnp.int32), ji])
                pltpu.sync_copy(wh_b.at[dv16],
                                rows.at[pl.ds(j * 16, 16)])
            for j in range(WW // 16):
                ji = j * 16 + iota
                lsv = plsc.load_gather(clsrc, [w * WW + ji])
                dv = plsc.load_gather(cdst, [jnp.full((16,), w, jnp.int32),
                                             ji])
                s1g = plsc.load_gather(s1_loc, [lsv])
                s2g = plsc.load_gather(s2_t, [dv])
                x = s1g + s2g
                xl = jnp.where(x >= 0, x, ALPHA * x)
                e = jnp.exp(-xl)
                gid = w * WW + ji
                e = jnp.where(gid < m_cnt, e, 0.0)
                e_buf[j * 16:(j + 1) * 16] = e

            def srow2(r, _):
                esp = plsc.load_gather(e_buf, [jnp.full((16,), r, jnp.int32)])
                lsp = plsc.load_gather(
                    clsrc, [jnp.full((16,), w * WW + r, jnp.int32)])
                hbase = lsp * H + iota
                for j in range(4):
                    rv = rows[r, j * 16:(j + 1) * 16]
                    plsc.addupdate_scatter(h_own, [hbase + j * 16], rv * esp)
                plsc.addupdate_scatter(rs_own, [lsp], esp, mask=mask0)
                return 0
            lax.fori_loop(0, WW, srow2, 0)
            return 0
        if _DBG_WIN:
            lax.fori_loop(0, nwin, win, 0)

        for k in range(ROWS_T // OCH):
            rb = lo + k * OCH
            pltpu.sync_copy(res_ref.at[b, c, pl.ds(rb, OCH)], nbuf_r)
            for j in range(OCH // 16):
                rsv = rs_own[k * OCH + j * 16:k * OCH + (j + 1) * 16]
                inv_buf[j * 16:(j + 1) * 16] = 1.0 / (rsv + EPS)
                rs_own[k * OCH + j * 16:k * OCH + (j + 1) * 16] = z16

            def orow(r, _):
                isp = plsc.load_gather(
                    inv_buf, [jnp.full((16,), r, jnp.int32)])
                hbase = (k * OCH + r) * H + iota
                for j in range(4):
                    hv = plsc.load_gather(h_own, [hbase + j * 16])
                    ov = hv * isp + nbuf_r[r, j * 16:(j + 1) * 16]
                    obuf[r, j * 16:(j + 1) * 16] = jnp.where(
                        ov >= 0, ov, ALPHA * ov)
                    plsc.store_scatter(h_own, [hbase + j * 16], z16)
                return 0
            lax.fori_loop(0, OCH, orow, 0)
            pltpu.sync_copy(obuf, out_ref.at[b, c, pl.ds(rb, OCH)])


_sc_call = functools.partial(
    pl.kernel,
    out_type=jax.ShapeDtypeStruct((B, 2, NP, H), jnp.float32),
    mesh=_mesh,
    compiler_params=pltpu.CompilerParams(needs_layout_passes=False,
                                         use_tc_tiling_on_sc=False),
    scratch_types=[
        pltpu.VMEM((SR, SW), jnp.int32),
        pltpu.VMEM((SR, SW), jnp.int32),
        pltpu.VMEM((CROWS * WW,), jnp.int32),
        pltpu.VMEM((CROWS, WW), jnp.int32),
        pltpu.VMEM((ROWS_T,), jnp.float32),
        pltpu.VMEM((NP,), jnp.float32),
        pltpu.VMEM((WW,), jnp.float32),
        pltpu.VMEM((WW, H), jnp.float32),
        pltpu.VMEM((ROWS_T * H,), jnp.float32),
        pltpu.VMEM((ROWS_T + 16,), jnp.float32),
        pltpu.VMEM((OCH, H), jnp.float32),
        pltpu.VMEM((OCH, H), jnp.float32),
        pltpu.VMEM((OCH,), jnp.float32),
    ],
)(_sc_body)


def kernel(X, edge_index, W_w, b_w, W_a, b_a, W_r, b_r):
    xp = jnp.pad(X, ((0, 0), (0, NP - N), (0, 0)))
    er = edge_index.reshape(2, NSTG, SR, SW)
    bw2 = b_w.reshape(1, D)
    br2 = b_r.reshape(1, D)
    ba2 = b_a.reshape(1, 1)
    wa2 = jnp.zeros((D, D), jnp.float32)
    wa2 = wa2.at[:, 0].set(W_a[0, :D]).at[:, 1].set(W_a[0, D:])
    wh, s, res = _tc_call(xp, W_w, bw2, wa2, ba2, W_r, br2)
    s_sq = s[..., 0]
    outp = _sc_call(er, wh, s_sq, res)
    return jnp.concatenate([outp[:, 0], outp[:, 1]], axis=-1)[:, :N]
